# baseline (speedup 1.0000x reference)
import functools

import jax
import jax.numpy as jnp
from jax.experimental import pallas as pl
from jax.experimental.pallas import tpu as pltpu


def _pick_bm(n: int) -> int:
    best = 8
    for bm in range(8, 257, 8):
        if n % bm == 0:
            best = bm
    return best


def _gcn_body(x_ref, adj_ref, w1_ref, b1_ref, w2_ref, b2_ref, out_ref,
              s_ref, h_ref, cache_ref, *, bm: int, k2: int):
    p = pl.program_id(0)
    m = pl.program_id(1)
    cached = ((m % 2) == 1) & (m < k2)

    @pl.when((p == 0) & (m == 0))
    def _():
        s1 = jnp.dot(x_ref[...].astype(jnp.bfloat16),
                     w1_ref[...].astype(jnp.bfloat16),
                     preferred_element_type=jnp.float32)
        s_ref[...] = s1.astype(jnp.bfloat16)

    @pl.when(p == 0)
    def _():
        adj_bf = adj_ref[...].astype(jnp.bfloat16)
        acc = jnp.dot(adj_bf, s_ref[...], preferred_element_type=jnp.float32)
        h = jnp.maximum(acc + b1_ref[...], 0.0)
        h_ref[pl.ds(m * bm, bm), :] = h.astype(jnp.bfloat16)

        @pl.when(cached)
        def _():
            cache_ref[pl.ds((m // 2) * bm, bm), :] = adj_bf

    @pl.when((p == 1) & (m == 0))
    def _():
        s2 = jnp.dot(h_ref[...], w2_ref[...].astype(jnp.bfloat16),
                     preferred_element_type=jnp.float32)
        s_ref[...] = s2.astype(jnp.bfloat16)

    @pl.when((p == 1) & jnp.logical_not(cached))
    def _():
        adj_bf = adj_ref[...].astype(jnp.bfloat16)
        acc = jnp.dot(adj_bf, s_ref[...], preferred_element_type=jnp.float32)
        out_ref[...] = acc + b2_ref[...]

    @pl.when((p == 1) & cached)
    def _():
        adj_bf = cache_ref[pl.ds((m // 2) * bm, bm), :]
        acc = jnp.dot(adj_bf, s_ref[...], preferred_element_type=jnp.float32)
        out_ref[...] = acc + b2_ref[...]


@jax.jit
def kernel(x, adj, W1, b1, W2, b2):
    n, nfeat = x.shape
    nhid = W1.shape[1]
    nout = W2.shape[1]
    bm = _pick_bm(n)
    g = n // bm
    k = min(6, g // 2)
    k2 = 2 * k

    b1r = b1.reshape(1, nhid)
    b2r = b2.reshape(1, nout)

    def adj_map(p, m):
        skip = (p == 1) & ((m % 2) == 1) & (m < k2)
        return (jnp.where(skip, m - 1, m), 0)

    return pl.pallas_call(
        functools.partial(_gcn_body, bm=bm, k2=k2),
        grid=(2, g),
        in_specs=[
            pl.BlockSpec((n, nfeat), lambda p, m: (0, 0)),
            pl.BlockSpec((bm, n), adj_map),
            pl.BlockSpec((nfeat, nhid), lambda p, m: (0, 0)),
            pl.BlockSpec((1, nhid), lambda p, m: (0, 0)),
            pl.BlockSpec((nhid, nout), lambda p, m: (0, 0)),
            pl.BlockSpec((1, nout), lambda p, m: (0, 0)),
        ],
        out_specs=pl.BlockSpec((bm, nout),
                               lambda p, m: (jnp.where(p == 1, m, 0), 0)),
        out_shape=jax.ShapeDtypeStruct((n, nout), jnp.float32),
        scratch_shapes=[
            pltpu.VMEM((n, nhid), jnp.bfloat16),
            pltpu.VMEM((n, nhid), jnp.bfloat16),
            pltpu.VMEM((max(k, 1) * bm, n), jnp.bfloat16),
        ],
        compiler_params=pltpu.CompilerParams(
            dimension_semantics=("arbitrary", "arbitrary"),
        ),
    )(x, adj, W1, b1r, W2, b2r)


# speedup vs baseline: 1.0339x; 1.0339x over previous
import functools

import jax
import jax.numpy as jnp
from jax.experimental import pallas as pl
from jax.experimental.pallas import tpu as pltpu


def _pick_bm(n: int) -> int:
    best = 8
    for bm in range(8, 513, 8):
        if n % bm == 0:
            best = bm
    return best


def _gcn_body(x_ref, adj_ref, w1_ref, b1_ref, w2_ref, b2_ref, out_ref,
              s_ref, h_ref, cache_ref, *, bm: int, k2: int):
    p = pl.program_id(0)
    m = pl.program_id(1)
    cached = ((m % 2) == 1) & (m < k2)

    @pl.when((p == 0) & (m == 0))
    def _():
        s1 = jnp.dot(x_ref[...].astype(jnp.bfloat16),
                     w1_ref[...].astype(jnp.bfloat16),
                     preferred_element_type=jnp.float32)
        s_ref[...] = s1.astype(jnp.bfloat16)

    @pl.when(p == 0)
    def _():
        adj_bf = adj_ref[...].astype(jnp.bfloat16)
        acc = jnp.dot(adj_bf, s_ref[...], preferred_element_type=jnp.float32)
        h = jnp.maximum(acc + b1_ref[...], 0.0)
        h_ref[pl.ds(m * bm, bm), :] = h.astype(jnp.bfloat16)

        @pl.when(cached)
        def _():
            cache_ref[pl.ds((m // 2) * bm, bm), :] = adj_bf

    @pl.when((p == 1) & (m == 0))
    def _():
        s2 = jnp.dot(h_ref[...], w2_ref[...].astype(jnp.bfloat16),
                     preferred_element_type=jnp.float32)
        s_ref[...] = s2.astype(jnp.bfloat16)

    @pl.when((p == 1) & jnp.logical_not(cached))
    def _():
        adj_bf = adj_ref[...].astype(jnp.bfloat16)
        acc = jnp.dot(adj_bf, s_ref[...], preferred_element_type=jnp.float32)
        out_ref[...] = acc + b2_ref[...]

    @pl.when((p == 1) & cached)
    def _():
        adj_bf = cache_ref[pl.ds((m // 2) * bm, bm), :]
        acc = jnp.dot(adj_bf, s_ref[...], preferred_element_type=jnp.float32)
        out_ref[...] = acc + b2_ref[...]


@jax.jit
def kernel(x, adj, W1, b1, W2, b2):
    n, nfeat = x.shape
    nhid = W1.shape[1]
    nout = W2.shape[1]
    bm = _pick_bm(n)
    g = n // bm
    k = min(0, g // 2)
    k2 = 2 * k

    b1r = b1.reshape(1, nhid)
    b2r = b2.reshape(1, nout)

    def adj_map(p, m):
        skip = (p == 1) & ((m % 2) == 1) & (m < k2)
        return (jnp.where(skip, m - 1, m), 0)

    return pl.pallas_call(
        functools.partial(_gcn_body, bm=bm, k2=k2),
        grid=(2, g),
        in_specs=[
            pl.BlockSpec((n, nfeat), lambda p, m: (0, 0)),
            pl.BlockSpec((bm, n), adj_map),
            pl.BlockSpec((nfeat, nhid), lambda p, m: (0, 0)),
            pl.BlockSpec((1, nhid), lambda p, m: (0, 0)),
            pl.BlockSpec((nhid, nout), lambda p, m: (0, 0)),
            pl.BlockSpec((1, nout), lambda p, m: (0, 0)),
        ],
        out_specs=pl.BlockSpec((bm, nout),
                               lambda p, m: (jnp.where(p == 1, m, 0), 0)),
        out_shape=jax.ShapeDtypeStruct((n, nout), jnp.float32),
        scratch_shapes=[
            pltpu.VMEM((n, nhid), jnp.bfloat16),
            pltpu.VMEM((n, nhid), jnp.bfloat16),
            pltpu.VMEM((max(k * bm, 8), n), jnp.bfloat16),
        ],
        compiler_params=pltpu.CompilerParams(
            dimension_semantics=("arbitrary", "arbitrary"),
        ),
    )(x, adj, W1, b1r, W2, b2r)
